# Initial kernel scaffold; baseline (speedup 1.0000x reference)
import jax
import jax.numpy as jnp
from jax.experimental import pallas as pl


def kernel(x, edge_index, Ws0, Ws1, Ws2, Ws3, Ws4, Ws5, Ws6, Ws7, Ws8, Ws9, Ws10, Ws11, Wn0, Wn1, Wn2, Wn3, Wn4, Wn5, Wn6, Wn7, Wn8, Wn9, Wn10, Wn11, g0, g1, g2, g3, g4, g5, g6, g7, g8, g9, b0, b1, b2, b3, b4, b5, b6, b7, b8, b9):
    raise NotImplementedError("write your pallas kernel here")


# speedup vs baseline: 8.8283x; 8.8283x over previous
import functools

import jax
import jax.numpy as jnp
from jax import lax
from jax.experimental import pallas as pl
from jax.experimental.pallas import tpu as pltpu
from jax.experimental.pallas import tpu_sc as plsc

N = 100000
NPAD = 100352
E = 1600000
EPAD = 1605632
CHUNK = 7168
CB = CHUNK // 128
RPS = NPAD // 16
BLK = 512
NBLK = NPAD // BLK
F32 = jnp.float32


def _make_sc(c, mode):
    nchunks = 7 if mode == "edge" else 14
    npass = 2 if mode == "cs4" else 1
    nslab = {"edge": 2, "cs": 2, "cs4": 4}[mode]
    mesh = plsc.VectorSubcoreMesh(core_axis_name="core", subcore_axis_name="sub")

    def body(table, srcp, dst2d, zrs, out, src_v, dst_v, rows_a, rows_b, acc,
             sem_a, sem_b):
        cr = lax.axis_index("core")
        sid = lax.axis_index("sub")
        my_rows = pl.multiple_of(sid * RPS, 128)
        if mode == "edge":
            base0 = (cr * 16 + sid) * (nchunks * CHUNK)
        else:
            base0 = sid * (nchunks * CHUNK)

        def run_pass(tbl, slab):
            pltpu.sync_copy(zrs, acc.at[pl.ds(my_rows, RPS)])
            plsc.subcore_barrier()

            def chunk_body(ch, carry):
                base = pl.multiple_of(base0 + ch * CHUNK, 128)
                pltpu.sync_copy(srcp.at[pl.ds(base, CHUNK)], src_v)
                pltpu.sync_copy(
                    dst2d.at[pl.ds(pl.multiple_of(base // 128, 8), CB)], dst_v)
                pltpu.async_copy(tbl.at[src_v.at[pl.ds(0, 128)]], rows_a, sem_a)

                def pair(j, c2):
                    b0 = 2 * j
                    b1 = b0 + 1
                    pltpu.async_copy(tbl.at[src_v.at[pl.ds(b1 * 128, 128)]],
                                     rows_b, sem_b)
                    pltpu.make_async_copy(tbl.at[src_v.at[pl.ds(b0 * 128, 128)]],
                                          rows_a, sem_a).wait()
                    pltpu.sync_copy(rows_a, acc.at[dst_v.at[b0]], add=True)

                    @pl.when(j < CB // 2 - 1)
                    def _():
                        pltpu.async_copy(
                            tbl.at[src_v.at[pl.ds((b0 + 2) * 128, 128)]],
                            rows_a, sem_a)

                    pltpu.make_async_copy(tbl.at[src_v.at[pl.ds(b1 * 128, 128)]],
                                          rows_b, sem_b).wait()
                    pltpu.sync_copy(rows_b, acc.at[dst_v.at[b1]], add=True)
                    return c2

                lax.fori_loop(0, CB // 2, pair, 0)
                return carry

            lax.fori_loop(0, nchunks, chunk_body, 0)
            plsc.subcore_barrier()
            pltpu.sync_copy(acc.at[pl.ds(my_rows, RPS)],
                            out.at[slab, pl.ds(my_rows, RPS)])

        if mode == "edge":
            run_pass(table, cr)
        elif mode == "cs":
            run_pass(table.at[cr], cr)
        else:
            for q in range(npass):
                run_pass(table.at[cr * 2 + q], cr * 2 + q)
                if q + 1 < npass:
                    plsc.subcore_barrier()

    return pl.kernel(
        body,
        out_type=jax.ShapeDtypeStruct((nslab, NPAD, c), F32),
        mesh=mesh,
        scratch_types=[
            pltpu.VMEM((CHUNK,), jnp.int32),
            pltpu.VMEM((CB, 128), jnp.int32),
            pltpu.VMEM((128, c), F32),
            pltpu.VMEM((128, c), F32),
            pltpu.VMEM_SHARED((NPAD, c), F32),
            pltpu.SemaphoreType.DMA,
            pltpu.SemaphoreType.DMA,
        ],
        compiler_params=pltpu.CompilerParams(use_tc_tiling_on_sc=False),
    )


def _t0(h, Wn, nsplit):
    cin, cout = Wn.shape

    if nsplit > 1:
        def body(h_ref, w_ref, o_ref):
            hv = h_ref[...]
            w = w_ref[...]
            for q in range(nsplit):
                o_ref[q] = jnp.dot(hv, w[:, 16 * q:16 * (q + 1)],
                                   preferred_element_type=F32)

        return pl.pallas_call(
            body,
            grid=(NBLK,),
            in_specs=[pl.BlockSpec((BLK, cin), lambda i: (i, 0)),
                      pl.BlockSpec((cin, cout), lambda i: (0, 0))],
            out_specs=pl.BlockSpec((nsplit, BLK, 16), lambda i: (0, i, 0)),
            out_shape=jax.ShapeDtypeStruct((nsplit, NPAD, 16), F32),
        )(h, Wn)

    def body(h_ref, w_ref, o_ref):
        o_ref[...] = jnp.dot(h_ref[...], w_ref[...],
                             preferred_element_type=F32)

    return pl.pallas_call(
        body,
        grid=(NBLK,),
        in_specs=[pl.BlockSpec((BLK, cin), lambda i: (i, 0)),
                  pl.BlockSpec((cin, cout), lambda i: (0, 0))],
        out_specs=pl.BlockSpec((BLK, cout), lambda i: (i, 0)),
        out_shape=jax.ShapeDtypeStruct((NPAD, cout), F32),
    )(h, Wn)


def _t3(parts0):
    def body(p_ref, o_ref):
        p = p_ref[...]
        d = p[0, :, 8:9] + p[1, :, 8:9]
        o_ref[...] = 1.0 / jnp.maximum(d, 1.0)

    return pl.pallas_call(
        body,
        grid=(NBLK,),
        in_specs=[pl.BlockSpec((2, BLK, 16), lambda i: (0, i, 0))],
        out_specs=pl.BlockSpec((BLK, 1), lambda i: (i, 0)),
        out_shape=jax.ShapeDtypeStruct((NPAD, 1), F32),
    )(parts0)


def _t1(parts, inv, h, Ws, mode, stats):
    cin, cout = Ws.shape
    nslab = parts.shape[0]
    ce = parts.shape[2]

    def body(p_ref, inv_ref, h_ref, ws_ref, *orefs):
        i = pl.program_id(0)
        p = p_ref[...]
        invv = inv_ref[...]
        if mode == "edge":
            pre = (p[0][:, :cout] + p[1][:, :cout]) * invv
        else:
            pre = jnp.concatenate([p[q] for q in range(nslab)], axis=1) * invv
        pre = pre + jnp.dot(h_ref[...], ws_ref[...],
                            preferred_element_type=F32)
        orefs[0][...] = pre
        if stats:
            s_ref = orefs[1]

            @pl.when(i == 0)
            def _():
                s_ref[...] = jnp.zeros_like(s_ref)

            ridx = i * BLK + lax.broadcasted_iota(jnp.int32, (BLK, 1), 0)
            m = (ridx < N).astype(F32)
            pm = pre * m
            s_ref[0:1, :] += jnp.sum(pm, axis=0, keepdims=True)
            s_ref[1:2, :] += jnp.sum(pre * pm, axis=0, keepdims=True)

    in_specs = [pl.BlockSpec((nslab, BLK, ce), lambda i: (0, i, 0)),
                pl.BlockSpec((BLK, 1), lambda i: (i, 0)),
                pl.BlockSpec((BLK, cin), lambda i: (i, 0)),
                pl.BlockSpec((cin, cout), lambda i: (0, 0))]
    out_specs = [pl.BlockSpec((BLK, cout), lambda i: (i, 0))]
    out_shape = [jax.ShapeDtypeStruct((NPAD, cout), F32)]
    if stats:
        out_specs.append(pl.BlockSpec((2, cout), lambda i: (0, 0)))
        out_shape.append(jax.ShapeDtypeStruct((2, cout), F32))

    res = pl.pallas_call(
        body,
        grid=(NBLK,),
        in_specs=in_specs,
        out_specs=tuple(out_specs) if stats else out_specs[0],
        out_shape=tuple(out_shape) if stats else out_shape[0],
    )(parts, inv, h, Ws)
    return res if stats else (res, None)


def _t2(pre, sums, g2, b2):
    cout = pre.shape[1]

    def body(pre_ref, s_ref, g_ref, b_ref, o_ref):
        s = s_ref[...]
        mu = s[0:1, :] * (1.0 / N)
        var = s[1:2, :] * (1.0 / N) - mu * mu
        scale = g_ref[...] * lax.rsqrt(var + 1e-5)
        shift = b_ref[...] - mu * scale
        o_ref[...] = jnp.maximum(pre_ref[...] * scale + shift, 0.0)

    return pl.pallas_call(
        body,
        grid=(NBLK,),
        in_specs=[pl.BlockSpec((BLK, cout), lambda i: (i, 0)),
                  pl.BlockSpec((2, cout), lambda i: (0, 0)),
                  pl.BlockSpec((1, cout), lambda i: (0, 0)),
                  pl.BlockSpec((1, cout), lambda i: (0, 0))],
        out_specs=pl.BlockSpec((BLK, cout), lambda i: (i, 0)),
        out_shape=jax.ShapeDtypeStruct((NPAD, cout), F32),
    )(pre, sums, g2, b2)


def kernel(x, edge_index,
           Ws0, Ws1, Ws2, Ws3, Ws4, Ws5, Ws6, Ws7, Ws8, Ws9, Ws10, Ws11,
           Wn0, Wn1, Wn2, Wn3, Wn4, Wn5, Wn6, Wn7, Wn8, Wn9, Wn10, Wn11,
           g0, g1, g2, g3, g4, g5, g6, g7, g8, g9,
           b0, b1, b2, b3, b4, b5, b6, b7, b8, b9):
    Ws = [Ws0, Ws1, Ws2, Ws3, Ws4, Ws5, Ws6, Ws7, Ws8, Ws9, Ws10, Ws11]
    Wn = [Wn0, Wn1, Wn2, Wn3, Wn4, Wn5, Wn6, Wn7, Wn8, Wn9, Wn10, Wn11]
    gl = [g0, g1, g2, g3, g4, g5, g6, g7, g8, g9]
    bl = [b0, b1, b2, b3, b4, b5, b6, b7, b8, b9]
    gl = [v.reshape(1, -1) for v in gl]
    bl = [v.reshape(1, -1) for v in bl]

    src = edge_index[0].astype(jnp.int32)
    dst = edge_index[1].astype(jnp.int32)
    srcp = jnp.concatenate([src, jnp.zeros((EPAD - E,), jnp.int32)])
    dstp = jnp.concatenate([dst, jnp.full((EPAD - E,), N, jnp.int32)])
    dst2d = dstp.reshape(EPAD // 128, 128)
    xp = jnp.pad(x, ((0, NPAD - N), (0, 0)))
    zc = {c: jnp.zeros((RPS, c), F32) for c in (8, 16)}

    sc_edge = {c: _make_sc(c, "edge") for c in (8, 16)}
    sc_cs = _make_sc(16, "cs")
    sc_cs4 = _make_sc(16, "cs4")
    bn_j = {ci: j for j, ci in enumerate([0, 1, 2, 3, 4, 5, 6, 8, 9, 10])}

    def conv(h, i, inv, extra_aug=False):
        cin, cout = Wn[i].shape
        if cout == 64:
            z = _t0(h, Wn[i], 4)
            p = sc_cs4(z, srcp, dst2d, zc[16])
            mode = "cs"
        elif cout == 32:
            z = _t0(h, Wn[i], 2)
            p = sc_cs(z, srcp, dst2d, zc[16])
            mode = "cs"
        else:
            if extra_aug:
                haug = jnp.concatenate([h, jnp.ones((NPAD, 1), F32)], axis=1)
                wn = jnp.zeros((cin + 1, 16), F32)
                wn = wn.at[:cin, :cout].set(Wn[i]).at[cin, 8].set(1.0)
                z = _t0(haug, wn, 1)
                p = sc_edge[16](z, srcp, dst2d, zc[16])
            else:
                cz = max(cout, 8)
                wn = (Wn[i] if cz == cout else
                      jnp.pad(Wn[i], ((0, 0), (0, cz - cout))))
                z = _t0(h, wn, 1)
                p = sc_edge[cz](z, srcp, dst2d, zc[cz])
            mode = "edge"
        if extra_aug:
            inv = _t3(p)
        want_bn = i in bn_j
        pre, s = _t1(p, inv, h, Ws[i], mode, want_bn)
        if want_bn:
            hn = _t2(pre, s, gl[bn_j[i]], bl[bn_j[i]])
        else:
            hn = pre
        return hn, inv

    h, inv = conv(xp, 0, None, extra_aug=True)
    for i in range(1, 4):
        h, _ = conv(h, i, inv)
    h3 = h
    r = h3
    for i in range(4, 8):
        r, _ = conv(r, i, inv)
    c_out = h3
    for i in range(8, 12):
        c_out, _ = conv(c_out, i, inv)
    return (c_out[:N], r[:N])


# speedup vs baseline: 9.4967x; 1.0757x over previous
import functools

import jax
import jax.numpy as jnp
from jax import lax
from jax.experimental import pallas as pl
from jax.experimental.pallas import tpu as pltpu
from jax.experimental.pallas import tpu_sc as plsc

N = 100000
NPAD = 100352
E = 1600000
EPAD = 1605632
CHUNK = 7168
CB = CHUNK // 128
RPS = NPAD // 16
BLK = 512
NBLK = NPAD // BLK
F32 = jnp.float32


def _make_sc(c, mode):
    nchunks = 7 if mode == "edge" else 14
    npass = 2 if mode == "cs4" else 1
    nslab = {"edge": 2, "cs": 2, "cs4": 4}[mode]
    mesh = plsc.VectorSubcoreMesh(core_axis_name="core", subcore_axis_name="sub")

    def body(table, srcp, dst2d, zrs, out, src_v, dst_v, rows_a, rows_b,
             rows_c, rows_d, acc, *sems):
        rows = [rows_a, rows_b, rows_c, rows_d]
        gsem = sems[:4]
        ssem = sems[4:]
        cr = lax.axis_index("core")
        sid = lax.axis_index("sub")
        my_rows = pl.multiple_of(sid * RPS, 128)
        if mode == "edge":
            base0 = (cr * 16 + sid) * (nchunks * CHUNK)
        else:
            base0 = sid * (nchunks * CHUNK)

        def run_pass(tbl, slab):
            pltpu.sync_copy(zrs, acc.at[pl.ds(my_rows, RPS)])
            plsc.subcore_barrier()

            def chunk_body(ch, carry):
                base = pl.multiple_of(base0 + ch * CHUNK, 128)
                pltpu.sync_copy(srcp.at[pl.ds(base, CHUNK)], src_v)
                pltpu.sync_copy(
                    dst2d.at[pl.ds(pl.multiple_of(base // 128, 8), CB)], dst_v)
                for b in range(4):
                    pltpu.async_copy(tbl.at[src_v.at[pl.ds(b * 128, 128)]],
                                     rows[b], gsem[b])

                def quad(j, c2):
                    for b in range(4):
                        bat = 4 * j + b
                        pltpu.make_async_copy(
                            tbl.at[src_v.at[pl.ds(bat * 128, 128)]],
                            rows[b], gsem[b]).wait()
                        pltpu.async_copy(rows[b], acc.at[dst_v.at[bat]],
                                         ssem[b], add=True)
                    for b in range(4):
                        pltpu.make_async_copy(rows[b],
                                              acc.at[dst_v.at[4 * j + b]],
                                              ssem[b]).wait()

                        @pl.when(j < CB // 4 - 1)
                        def _():
                            pltpu.async_copy(
                                tbl.at[src_v.at[pl.ds((4 * j + b + 4) * 128,
                                                      128)]],
                                rows[b], gsem[b])
                    return c2

                lax.fori_loop(0, CB // 4, quad, 0)
                return carry

            lax.fori_loop(0, nchunks, chunk_body, 0)
            plsc.subcore_barrier()
            pltpu.sync_copy(acc.at[pl.ds(my_rows, RPS)],
                            out.at[slab, pl.ds(my_rows, RPS)])

        if mode == "edge":
            run_pass(table, cr)
        elif mode == "cs":
            run_pass(table.at[cr], cr)
        else:
            for q in range(npass):
                run_pass(table.at[cr * 2 + q], cr * 2 + q)
                if q + 1 < npass:
                    plsc.subcore_barrier()

    return pl.kernel(
        body,
        out_type=jax.ShapeDtypeStruct((nslab, NPAD, c), F32),
        mesh=mesh,
        scratch_types=[
            pltpu.VMEM((CHUNK,), jnp.int32),
            pltpu.VMEM((CB, 128), jnp.int32),
            pltpu.VMEM((128, c), F32),
            pltpu.VMEM((128, c), F32),
            pltpu.VMEM((128, c), F32),
            pltpu.VMEM((128, c), F32),
            pltpu.VMEM_SHARED((NPAD, c), F32),
        ] + [pltpu.SemaphoreType.DMA] * 8,
        compiler_params=pltpu.CompilerParams(use_tc_tiling_on_sc=False),
    )


def _t0(h, Wn, nsplit):
    cin, cout = Wn.shape

    if nsplit > 1:
        def body(h_ref, w_ref, o_ref):
            hv = h_ref[...]
            w = w_ref[...]
            for q in range(nsplit):
                o_ref[q] = jnp.dot(hv, w[:, 16 * q:16 * (q + 1)],
                                   preferred_element_type=F32)

        return pl.pallas_call(
            body,
            grid=(NBLK,),
            in_specs=[pl.BlockSpec((BLK, cin), lambda i: (i, 0)),
                      pl.BlockSpec((cin, cout), lambda i: (0, 0))],
            out_specs=pl.BlockSpec((nsplit, BLK, 16), lambda i: (0, i, 0)),
            out_shape=jax.ShapeDtypeStruct((nsplit, NPAD, 16), F32),
        )(h, Wn)

    def body(h_ref, w_ref, o_ref):
        o_ref[...] = jnp.dot(h_ref[...], w_ref[...],
                             preferred_element_type=F32)

    return pl.pallas_call(
        body,
        grid=(NBLK,),
        in_specs=[pl.BlockSpec((BLK, cin), lambda i: (i, 0)),
                  pl.BlockSpec((cin, cout), lambda i: (0, 0))],
        out_specs=pl.BlockSpec((BLK, cout), lambda i: (i, 0)),
        out_shape=jax.ShapeDtypeStruct((NPAD, cout), F32),
    )(h, Wn)


def _t3(parts0):
    def body(p_ref, o_ref):
        p = p_ref[...]
        d = p[0, :, 8:9] + p[1, :, 8:9]
        o_ref[...] = 1.0 / jnp.maximum(d, 1.0)

    return pl.pallas_call(
        body,
        grid=(NBLK,),
        in_specs=[pl.BlockSpec((2, BLK, 16), lambda i: (0, i, 0))],
        out_specs=pl.BlockSpec((BLK, 1), lambda i: (i, 0)),
        out_shape=jax.ShapeDtypeStruct((NPAD, 1), F32),
    )(parts0)


def _t1(parts, inv, h, Ws, mode, stats):
    cin, cout = Ws.shape
    nslab = parts.shape[0]
    ce = parts.shape[2]

    def body(p_ref, inv_ref, h_ref, ws_ref, *orefs):
        i = pl.program_id(0)
        p = p_ref[...]
        invv = inv_ref[...]
        if mode == "edge":
            pre = (p[0][:, :cout] + p[1][:, :cout]) * invv
        else:
            pre = jnp.concatenate([p[q] for q in range(nslab)], axis=1) * invv
        pre = pre + jnp.dot(h_ref[...], ws_ref[...],
                            preferred_element_type=F32)
        orefs[0][...] = pre
        if stats:
            s_ref = orefs[1]

            @pl.when(i == 0)
            def _():
                s_ref[...] = jnp.zeros_like(s_ref)

            ridx = i * BLK + lax.broadcasted_iota(jnp.int32, (BLK, 1), 0)
            m = (ridx < N).astype(F32)
            pm = pre * m
            s_ref[0:1, :] += jnp.sum(pm, axis=0, keepdims=True)
            s_ref[1:2, :] += jnp.sum(pre * pm, axis=0, keepdims=True)

    in_specs = [pl.BlockSpec((nslab, BLK, ce), lambda i: (0, i, 0)),
                pl.BlockSpec((BLK, 1), lambda i: (i, 0)),
                pl.BlockSpec((BLK, cin), lambda i: (i, 0)),
                pl.BlockSpec((cin, cout), lambda i: (0, 0))]
    out_specs = [pl.BlockSpec((BLK, cout), lambda i: (i, 0))]
    out_shape = [jax.ShapeDtypeStruct((NPAD, cout), F32)]
    if stats:
        out_specs.append(pl.BlockSpec((2, cout), lambda i: (0, 0)))
        out_shape.append(jax.ShapeDtypeStruct((2, cout), F32))

    res = pl.pallas_call(
        body,
        grid=(NBLK,),
        in_specs=in_specs,
        out_specs=tuple(out_specs) if stats else out_specs[0],
        out_shape=tuple(out_shape) if stats else out_shape[0],
    )(parts, inv, h, Ws)
    return res if stats else (res, None)


def _t2(pre, sums, g2, b2):
    cout = pre.shape[1]

    def body(pre_ref, s_ref, g_ref, b_ref, o_ref):
        s = s_ref[...]
        mu = s[0:1, :] * (1.0 / N)
        var = s[1:2, :] * (1.0 / N) - mu * mu
        scale = g_ref[...] * lax.rsqrt(var + 1e-5)
        shift = b_ref[...] - mu * scale
        o_ref[...] = jnp.maximum(pre_ref[...] * scale + shift, 0.0)

    return pl.pallas_call(
        body,
        grid=(NBLK,),
        in_specs=[pl.BlockSpec((BLK, cout), lambda i: (i, 0)),
                  pl.BlockSpec((2, cout), lambda i: (0, 0)),
                  pl.BlockSpec((1, cout), lambda i: (0, 0)),
                  pl.BlockSpec((1, cout), lambda i: (0, 0))],
        out_specs=pl.BlockSpec((BLK, cout), lambda i: (i, 0)),
        out_shape=jax.ShapeDtypeStruct((NPAD, cout), F32),
    )(pre, sums, g2, b2)


def kernel(x, edge_index,
           Ws0, Ws1, Ws2, Ws3, Ws4, Ws5, Ws6, Ws7, Ws8, Ws9, Ws10, Ws11,
           Wn0, Wn1, Wn2, Wn3, Wn4, Wn5, Wn6, Wn7, Wn8, Wn9, Wn10, Wn11,
           g0, g1, g2, g3, g4, g5, g6, g7, g8, g9,
           b0, b1, b2, b3, b4, b5, b6, b7, b8, b9):
    Ws = [Ws0, Ws1, Ws2, Ws3, Ws4, Ws5, Ws6, Ws7, Ws8, Ws9, Ws10, Ws11]
    Wn = [Wn0, Wn1, Wn2, Wn3, Wn4, Wn5, Wn6, Wn7, Wn8, Wn9, Wn10, Wn11]
    gl = [g0, g1, g2, g3, g4, g5, g6, g7, g8, g9]
    bl = [b0, b1, b2, b3, b4, b5, b6, b7, b8, b9]
    gl = [v.reshape(1, -1) for v in gl]
    bl = [v.reshape(1, -1) for v in bl]

    src = edge_index[0].astype(jnp.int32)
    dst = edge_index[1].astype(jnp.int32)
    srcp = jnp.concatenate([src, jnp.zeros((EPAD - E,), jnp.int32)])
    dstp = jnp.concatenate([dst, jnp.full((EPAD - E,), N, jnp.int32)])
    dst2d = dstp.reshape(EPAD // 128, 128)
    xp = jnp.pad(x, ((0, NPAD - N), (0, 0)))
    zc = {c: jnp.zeros((RPS, c), F32) for c in (8, 16)}

    sc_edge = {c: _make_sc(c, "edge") for c in (8, 16)}
    sc_cs = _make_sc(16, "cs")
    sc_cs4 = _make_sc(16, "cs4")
    bn_j = {ci: j for j, ci in enumerate([0, 1, 2, 3, 4, 5, 6, 8, 9, 10])}

    def conv(h, i, inv, extra_aug=False):
        cin, cout = Wn[i].shape
        if cout == 64:
            z = _t0(h, Wn[i], 4)
            p = sc_cs4(z, srcp, dst2d, zc[16])
            mode = "cs"
        elif cout == 32:
            z = _t0(h, Wn[i], 2)
            p = sc_cs(z, srcp, dst2d, zc[16])
            mode = "cs"
        else:
            if extra_aug:
                haug = jnp.concatenate([h, jnp.ones((NPAD, 1), F32)], axis=1)
                wn = jnp.zeros((cin + 1, 16), F32)
                wn = wn.at[:cin, :cout].set(Wn[i]).at[cin, 8].set(1.0)
                z = _t0(haug, wn, 1)
                p = sc_edge[16](z, srcp, dst2d, zc[16])
            else:
                cz = max(cout, 8)
                wn = (Wn[i] if cz == cout else
                      jnp.pad(Wn[i], ((0, 0), (0, cz - cout))))
                z = _t0(h, wn, 1)
                p = sc_edge[cz](z, srcp, dst2d, zc[cz])
            mode = "edge"
        if extra_aug:
            inv = _t3(p)
        want_bn = i in bn_j
        pre, s = _t1(p, inv, h, Ws[i], mode, want_bn)
        if want_bn:
            hn = _t2(pre, s, gl[bn_j[i]], bl[bn_j[i]])
        else:
            hn = pre
        return hn, inv

    h, inv = conv(xp, 0, None, extra_aug=True)
    for i in range(1, 4):
        h, _ = conv(h, i, inv)
    h3 = h
    r = h3
    for i in range(4, 8):
        r, _ = conv(r, i, inv)
    c_out = h3
    for i in range(8, 12):
        c_out, _ = conv(c_out, i, inv)
    return (c_out[:N], r[:N])


# speedup vs baseline: 9.8163x; 1.0337x over previous
import functools

import jax
import jax.numpy as jnp
from jax import lax
from jax.experimental import pallas as pl
from jax.experimental.pallas import tpu as pltpu
from jax.experimental.pallas import tpu_sc as plsc

N = 100000
NPAD = 100352
E = 1600000
EPAD = 1605632
CHUNK = 7168
CB = CHUNK // 128
RPS = NPAD // 16
BLK = 512
NBLK = NPAD // BLK
F32 = jnp.float32


def _make_sc(c, mode):
    nchunks = 7 if mode == "edge" else 14
    npass = 2 if mode == "cs4" else 1
    nslab = {"edge": 2, "cs": 2, "cs4": 4}[mode]
    mesh = plsc.VectorSubcoreMesh(core_axis_name="core", subcore_axis_name="sub")

    def body(table, srcp, dst2d, zrs, out, src_v, dst_v, *rest):
        rows = list(rest[:8])
        acc = rest[8]
        sems = rest[9:]
        gsem = sems[:8]
        ssem = sems[8:]
        cr = lax.axis_index("core")
        sid = lax.axis_index("sub")
        my_rows = pl.multiple_of(sid * RPS, 128)
        if mode == "edge":
            base0 = (cr * 16 + sid) * (nchunks * CHUNK)
        else:
            base0 = sid * (nchunks * CHUNK)

        def run_pass(tbl, slab):
            pltpu.sync_copy(zrs, acc.at[pl.ds(my_rows, RPS)])
            plsc.subcore_barrier()

            def chunk_body(ch, carry):
                base = pl.multiple_of(base0 + ch * CHUNK, 128)
                pltpu.sync_copy(srcp.at[pl.ds(base, CHUNK)], src_v)
                pltpu.sync_copy(
                    dst2d.at[pl.ds(pl.multiple_of(base // 128, 8), CB)], dst_v)
                for b in range(8):
                    pltpu.async_copy(tbl.at[src_v.at[pl.ds(b * 128, 128)]],
                                     rows[b], gsem[b])

                def octet(j, c2):
                    for b in range(8):
                        bat = 8 * j + b
                        pltpu.make_async_copy(
                            tbl.at[src_v.at[pl.ds(bat * 128, 128)]],
                            rows[b], gsem[b]).wait()
                        pltpu.async_copy(rows[b], acc.at[dst_v.at[bat]],
                                         ssem[b], add=True)
                    for b in range(8):
                        pltpu.make_async_copy(rows[b],
                                              acc.at[dst_v.at[8 * j + b]],
                                              ssem[b]).wait()

                        @pl.when(j < CB // 8 - 1)
                        def _():
                            pltpu.async_copy(
                                tbl.at[src_v.at[pl.ds((8 * j + b + 8) * 128,
                                                      128)]],
                                rows[b], gsem[b])
                    return c2

                lax.fori_loop(0, CB // 8, octet, 0)
                return carry

            lax.fori_loop(0, nchunks, chunk_body, 0)
            plsc.subcore_barrier()
            pltpu.sync_copy(acc.at[pl.ds(my_rows, RPS)],
                            out.at[slab, pl.ds(my_rows, RPS)])

        if mode == "edge":
            run_pass(table, cr)
        elif mode == "cs":
            run_pass(table.at[cr], cr)
        else:
            for q in range(npass):
                run_pass(table.at[cr * 2 + q], cr * 2 + q)
                if q + 1 < npass:
                    plsc.subcore_barrier()

    return pl.kernel(
        body,
        out_type=jax.ShapeDtypeStruct((nslab, NPAD, c), F32),
        mesh=mesh,
        scratch_types=[
            pltpu.VMEM((CHUNK,), jnp.int32),
            pltpu.VMEM((CB, 128), jnp.int32),
        ] + [pltpu.VMEM((128, c), F32)] * 8 + [
            pltpu.VMEM_SHARED((NPAD, c), F32),
        ] + [pltpu.SemaphoreType.DMA] * 16,
        compiler_params=pltpu.CompilerParams(use_tc_tiling_on_sc=False),
    )


def _t0(h, Wn, nsplit):
    cin, cout = Wn.shape

    if nsplit > 1:
        def body(h_ref, w_ref, o_ref):
            hv = h_ref[...]
            w = w_ref[...]
            for q in range(nsplit):
                o_ref[q] = jnp.dot(hv, w[:, 16 * q:16 * (q + 1)],
                                   preferred_element_type=F32)

        return pl.pallas_call(
            body,
            grid=(NBLK,),
            in_specs=[pl.BlockSpec((BLK, cin), lambda i: (i, 0)),
                      pl.BlockSpec((cin, cout), lambda i: (0, 0))],
            out_specs=pl.BlockSpec((nsplit, BLK, 16), lambda i: (0, i, 0)),
            out_shape=jax.ShapeDtypeStruct((nsplit, NPAD, 16), F32),
        )(h, Wn)

    def body(h_ref, w_ref, o_ref):
        o_ref[...] = jnp.dot(h_ref[...], w_ref[...],
                             preferred_element_type=F32)

    return pl.pallas_call(
        body,
        grid=(NBLK,),
        in_specs=[pl.BlockSpec((BLK, cin), lambda i: (i, 0)),
                  pl.BlockSpec((cin, cout), lambda i: (0, 0))],
        out_specs=pl.BlockSpec((BLK, cout), lambda i: (i, 0)),
        out_shape=jax.ShapeDtypeStruct((NPAD, cout), F32),
    )(h, Wn)


def _t3(parts0):
    def body(p_ref, o_ref):
        p = p_ref[...]
        d = p[0, :, 8:9] + p[1, :, 8:9]
        o_ref[...] = 1.0 / jnp.maximum(d, 1.0)

    return pl.pallas_call(
        body,
        grid=(NBLK,),
        in_specs=[pl.BlockSpec((2, BLK, 16), lambda i: (0, i, 0))],
        out_specs=pl.BlockSpec((BLK, 1), lambda i: (i, 0)),
        out_shape=jax.ShapeDtypeStruct((NPAD, 1), F32),
    )(parts0)


def _t1(parts, inv, h, Ws, mode, stats):
    cin, cout = Ws.shape
    nslab = parts.shape[0]
    ce = parts.shape[2]

    def body(p_ref, inv_ref, h_ref, ws_ref, *orefs):
        i = pl.program_id(0)
        p = p_ref[...]
        invv = inv_ref[...]
        if mode == "edge":
            pre = (p[0][:, :cout] + p[1][:, :cout]) * invv
        else:
            pre = jnp.concatenate([p[q] for q in range(nslab)], axis=1) * invv
        pre = pre + jnp.dot(h_ref[...], ws_ref[...],
                            preferred_element_type=F32)
        orefs[0][...] = pre
        if stats:
            s_ref = orefs[1]

            @pl.when(i == 0)
            def _():
                s_ref[...] = jnp.zeros_like(s_ref)

            ridx = i * BLK + lax.broadcasted_iota(jnp.int32, (BLK, 1), 0)
            m = (ridx < N).astype(F32)
            pm = pre * m
            s_ref[0:1, :] += jnp.sum(pm, axis=0, keepdims=True)
            s_ref[1:2, :] += jnp.sum(pre * pm, axis=0, keepdims=True)

    in_specs = [pl.BlockSpec((nslab, BLK, ce), lambda i: (0, i, 0)),
                pl.BlockSpec((BLK, 1), lambda i: (i, 0)),
                pl.BlockSpec((BLK, cin), lambda i: (i, 0)),
                pl.BlockSpec((cin, cout), lambda i: (0, 0))]
    out_specs = [pl.BlockSpec((BLK, cout), lambda i: (i, 0))]
    out_shape = [jax.ShapeDtypeStruct((NPAD, cout), F32)]
    if stats:
        out_specs.append(pl.BlockSpec((2, cout), lambda i: (0, 0)))
        out_shape.append(jax.ShapeDtypeStruct((2, cout), F32))

    res = pl.pallas_call(
        body,
        grid=(NBLK,),
        in_specs=in_specs,
        out_specs=tuple(out_specs) if stats else out_specs[0],
        out_shape=tuple(out_shape) if stats else out_shape[0],
    )(parts, inv, h, Ws)
    return res if stats else (res, None)


def _t2(pre, sums, g2, b2):
    cout = pre.shape[1]

    def body(pre_ref, s_ref, g_ref, b_ref, o_ref):
        s = s_ref[...]
        mu = s[0:1, :] * (1.0 / N)
        var = s[1:2, :] * (1.0 / N) - mu * mu
        scale = g_ref[...] * lax.rsqrt(var + 1e-5)
        shift = b_ref[...] - mu * scale
        o_ref[...] = jnp.maximum(pre_ref[...] * scale + shift, 0.0)

    return pl.pallas_call(
        body,
        grid=(NBLK,),
        in_specs=[pl.BlockSpec((BLK, cout), lambda i: (i, 0)),
                  pl.BlockSpec((2, cout), lambda i: (0, 0)),
                  pl.BlockSpec((1, cout), lambda i: (0, 0)),
                  pl.BlockSpec((1, cout), lambda i: (0, 0))],
        out_specs=pl.BlockSpec((BLK, cout), lambda i: (i, 0)),
        out_shape=jax.ShapeDtypeStruct((NPAD, cout), F32),
    )(pre, sums, g2, b2)


def kernel(x, edge_index,
           Ws0, Ws1, Ws2, Ws3, Ws4, Ws5, Ws6, Ws7, Ws8, Ws9, Ws10, Ws11,
           Wn0, Wn1, Wn2, Wn3, Wn4, Wn5, Wn6, Wn7, Wn8, Wn9, Wn10, Wn11,
           g0, g1, g2, g3, g4, g5, g6, g7, g8, g9,
           b0, b1, b2, b3, b4, b5, b6, b7, b8, b9):
    Ws = [Ws0, Ws1, Ws2, Ws3, Ws4, Ws5, Ws6, Ws7, Ws8, Ws9, Ws10, Ws11]
    Wn = [Wn0, Wn1, Wn2, Wn3, Wn4, Wn5, Wn6, Wn7, Wn8, Wn9, Wn10, Wn11]
    gl = [g0, g1, g2, g3, g4, g5, g6, g7, g8, g9]
    bl = [b0, b1, b2, b3, b4, b5, b6, b7, b8, b9]
    gl = [v.reshape(1, -1) for v in gl]
    bl = [v.reshape(1, -1) for v in bl]

    src = edge_index[0].astype(jnp.int32)
    dst = edge_index[1].astype(jnp.int32)
    srcp = jnp.concatenate([src, jnp.zeros((EPAD - E,), jnp.int32)])
    dstp = jnp.concatenate([dst, jnp.full((EPAD - E,), N, jnp.int32)])
    dst2d = dstp.reshape(EPAD // 128, 128)
    xp = jnp.pad(x, ((0, NPAD - N), (0, 0)))
    zc = {c: jnp.zeros((RPS, c), F32) for c in (8, 16)}

    sc_edge = {c: _make_sc(c, "edge") for c in (8, 16)}
    sc_cs = _make_sc(16, "cs")
    sc_cs4 = _make_sc(16, "cs4")
    bn_j = {ci: j for j, ci in enumerate([0, 1, 2, 3, 4, 5, 6, 8, 9, 10])}

    def conv(h, i, inv, extra_aug=False):
        cin, cout = Wn[i].shape
        if cout == 64:
            z = _t0(h, Wn[i], 4)
            p = sc_cs4(z, srcp, dst2d, zc[16])
            mode = "cs"
        elif cout == 32:
            z = _t0(h, Wn[i], 2)
            p = sc_cs(z, srcp, dst2d, zc[16])
            mode = "cs"
        else:
            if extra_aug:
                haug = jnp.concatenate([h, jnp.ones((NPAD, 1), F32)], axis=1)
                wn = jnp.zeros((cin + 1, 16), F32)
                wn = wn.at[:cin, :cout].set(Wn[i]).at[cin, 8].set(1.0)
                z = _t0(haug, wn, 1)
                p = sc_edge[16](z, srcp, dst2d, zc[16])
            else:
                cz = max(cout, 8)
                wn = (Wn[i] if cz == cout else
                      jnp.pad(Wn[i], ((0, 0), (0, cz - cout))))
                z = _t0(h, wn, 1)
                p = sc_edge[cz](z, srcp, dst2d, zc[cz])
            mode = "edge"
        if extra_aug:
            inv = _t3(p)
        want_bn = i in bn_j
        pre, s = _t1(p, inv, h, Ws[i], mode, want_bn)
        if want_bn:
            hn = _t2(pre, s, gl[bn_j[i]], bl[bn_j[i]])
        else:
            hn = pre
        return hn, inv

    h, inv = conv(xp, 0, None, extra_aug=True)
    for i in range(1, 4):
        h, _ = conv(h, i, inv)
    h3 = h
    r = h3
    for i in range(4, 8):
        r, _ = conv(r, i, inv)
    c_out = h3
    for i in range(8, 12):
        c_out, _ = conv(c_out, i, inv)
    return (c_out[:N], r[:N])


# speedup vs baseline: 14.1657x; 1.4431x over previous
import functools

import jax
import jax.numpy as jnp
from jax import lax
from jax.experimental import pallas as pl
from jax.experimental.pallas import tpu as pltpu
from jax.experimental.pallas import tpu_sc as plsc

N = 100000
NPAD = 100352
E = 1600000
EPAD = 1605632
CHUNK = 7168
CB = CHUNK // 128
RPS = NPAD // 16
BLK = 2048
NBLK = NPAD // BLK
F32 = jnp.float32


def _make_sc(c, mode):
    nchunks = 7 if mode == "edge" else 14
    npass = 2 if mode == "cs4" else 1
    nslab = {"edge": 2, "cs": 2, "cs4": 4}[mode]
    mesh = plsc.VectorSubcoreMesh(core_axis_name="core", subcore_axis_name="sub")

    def body(table, srcp, dst2d, zrs, out, src_v, dst_v, *rest):
        rows = list(rest[:8])
        acc = rest[8]
        sems = rest[9:]
        gsem = sems[:8]
        ssem = sems[8:]
        cr = lax.axis_index("core")
        sid = lax.axis_index("sub")
        my_rows = pl.multiple_of(sid * RPS, 128)
        if mode == "edge":
            base0 = (cr * 16 + sid) * (nchunks * CHUNK)
        else:
            base0 = sid * (nchunks * CHUNK)

        def run_pass(tbl, slab):
            pltpu.sync_copy(zrs, acc.at[pl.ds(my_rows, RPS)])
            plsc.subcore_barrier()

            def chunk_body(ch, carry):
                base = pl.multiple_of(base0 + ch * CHUNK, 128)
                pltpu.sync_copy(srcp.at[pl.ds(base, CHUNK)], src_v)
                pltpu.sync_copy(
                    dst2d.at[pl.ds(pl.multiple_of(base // 128, 8), CB)], dst_v)
                for b in range(8):
                    pltpu.async_copy(tbl.at[src_v.at[pl.ds(b * 128, 128)]],
                                     rows[b], gsem[b])

                def octet(j, c2):
                    for b in range(8):
                        bat = 8 * j + b
                        pltpu.make_async_copy(
                            tbl.at[src_v.at[pl.ds(bat * 128, 128)]],
                            rows[b], gsem[b]).wait()
                        pltpu.async_copy(rows[b], acc.at[dst_v.at[bat]],
                                         ssem[b], add=True)
                    for b in range(8):
                        pltpu.make_async_copy(rows[b],
                                              acc.at[dst_v.at[8 * j + b]],
                                              ssem[b]).wait()

                        @pl.when(j < CB // 8 - 1)
                        def _():
                            pltpu.async_copy(
                                tbl.at[src_v.at[pl.ds((8 * j + b + 8) * 128,
                                                      128)]],
                                rows[b], gsem[b])
                    return c2

                lax.fori_loop(0, CB // 8, octet, 0)
                return carry

            lax.fori_loop(0, nchunks, chunk_body, 0)
            plsc.subcore_barrier()
            pltpu.sync_copy(acc.at[pl.ds(my_rows, RPS)],
                            out.at[slab, pl.ds(my_rows, RPS)])

        if mode == "edge":
            run_pass(table, cr)
        elif mode == "cs":
            run_pass(table.at[cr], cr)
        else:
            for q in range(npass):
                run_pass(table.at[cr * 2 + q], cr * 2 + q)
                if q + 1 < npass:
                    plsc.subcore_barrier()

    return pl.kernel(
        body,
        out_type=jax.ShapeDtypeStruct((nslab, NPAD, c), F32),
        mesh=mesh,
        scratch_types=[
            pltpu.VMEM((CHUNK,), jnp.int32),
            pltpu.VMEM((CB, 128), jnp.int32),
        ] + [pltpu.VMEM((128, c), F32)] * 8 + [
            pltpu.VMEM_SHARED((NPAD, c), F32),
        ] + [pltpu.SemaphoreType.DMA] * 16,
        compiler_params=pltpu.CompilerParams(use_tc_tiling_on_sc=False),
    )


def _t0(h, Wn, nsplit):
    cin, cout = Wn.shape

    if nsplit > 1:
        def body(h_ref, w_ref, o_ref):
            hv = h_ref[...]
            w = w_ref[...]
            for q in range(nsplit):
                o_ref[q] = jnp.dot(hv, w[:, 16 * q:16 * (q + 1)],
                                   preferred_element_type=F32)

        return pl.pallas_call(
            body,
            grid=(NBLK,),
            in_specs=[pl.BlockSpec((BLK, cin), lambda i: (i, 0)),
                      pl.BlockSpec((cin, cout), lambda i: (0, 0))],
            out_specs=pl.BlockSpec((nsplit, BLK, 16), lambda i: (0, i, 0)),
            out_shape=jax.ShapeDtypeStruct((nsplit, NPAD, 16), F32),
        )(h, Wn)

    def body(h_ref, w_ref, o_ref):
        o_ref[...] = jnp.dot(h_ref[...], w_ref[...],
                             preferred_element_type=F32)

    return pl.pallas_call(
        body,
        grid=(NBLK,),
        in_specs=[pl.BlockSpec((BLK, cin), lambda i: (i, 0)),
                  pl.BlockSpec((cin, cout), lambda i: (0, 0))],
        out_specs=pl.BlockSpec((BLK, cout), lambda i: (i, 0)),
        out_shape=jax.ShapeDtypeStruct((NPAD, cout), F32),
    )(h, Wn)


def _t3(parts0):
    def body(p_ref, o_ref):
        p = p_ref[...]
        d = p[0, :, 8:9] + p[1, :, 8:9]
        o_ref[...] = 1.0 / jnp.maximum(d, 1.0)

    return pl.pallas_call(
        body,
        grid=(NBLK,),
        in_specs=[pl.BlockSpec((2, BLK, 16), lambda i: (0, i, 0))],
        out_specs=pl.BlockSpec((BLK, 1), lambda i: (i, 0)),
        out_shape=jax.ShapeDtypeStruct((NPAD, 1), F32),
    )(parts0)


def _t1(parts, inv, h, Ws, mode, stats):
    cin, cout = Ws.shape
    nslab = parts.shape[0]
    ce = parts.shape[2]

    def body(p_ref, inv_ref, h_ref, ws_ref, *orefs):
        i = pl.program_id(0)
        p = p_ref[...]
        invv = inv_ref[...]
        if mode == "edge":
            pre = (p[0][:, :cout] + p[1][:, :cout]) * invv
        else:
            pre = jnp.concatenate([p[q] for q in range(nslab)], axis=1) * invv
        pre = pre + jnp.dot(h_ref[...], ws_ref[...],
                            preferred_element_type=F32)
        orefs[0][...] = pre
        if stats:
            s_ref = orefs[1]

            @pl.when(i == 0)
            def _():
                s_ref[...] = jnp.zeros_like(s_ref)

            ridx = i * BLK + lax.broadcasted_iota(jnp.int32, (BLK, 1), 0)
            m = (ridx < N).astype(F32)
            pm = pre * m
            s_ref[0:1, :] += jnp.sum(pm, axis=0, keepdims=True)
            s_ref[1:2, :] += jnp.sum(pre * pm, axis=0, keepdims=True)

    in_specs = [pl.BlockSpec((nslab, BLK, ce), lambda i: (0, i, 0)),
                pl.BlockSpec((BLK, 1), lambda i: (i, 0)),
                pl.BlockSpec((BLK, cin), lambda i: (i, 0)),
                pl.BlockSpec((cin, cout), lambda i: (0, 0))]
    out_specs = [pl.BlockSpec((BLK, cout), lambda i: (i, 0))]
    out_shape = [jax.ShapeDtypeStruct((NPAD, cout), F32)]
    if stats:
        out_specs.append(pl.BlockSpec((2, cout), lambda i: (0, 0)))
        out_shape.append(jax.ShapeDtypeStruct((2, cout), F32))

    res = pl.pallas_call(
        body,
        grid=(NBLK,),
        in_specs=in_specs,
        out_specs=tuple(out_specs) if stats else out_specs[0],
        out_shape=tuple(out_shape) if stats else out_shape[0],
    )(parts, inv, h, Ws)
    return res if stats else (res, None)


def _t2(pre, sums, g2, b2):
    cout = pre.shape[1]

    def body(pre_ref, s_ref, g_ref, b_ref, o_ref):
        s = s_ref[...]
        mu = s[0:1, :] * (1.0 / N)
        var = s[1:2, :] * (1.0 / N) - mu * mu
        scale = g_ref[...] * lax.rsqrt(var + 1e-5)
        shift = b_ref[...] - mu * scale
        o_ref[...] = jnp.maximum(pre_ref[...] * scale + shift, 0.0)

    return pl.pallas_call(
        body,
        grid=(NBLK,),
        in_specs=[pl.BlockSpec((BLK, cout), lambda i: (i, 0)),
                  pl.BlockSpec((2, cout), lambda i: (0, 0)),
                  pl.BlockSpec((1, cout), lambda i: (0, 0)),
                  pl.BlockSpec((1, cout), lambda i: (0, 0))],
        out_specs=pl.BlockSpec((BLK, cout), lambda i: (i, 0)),
        out_shape=jax.ShapeDtypeStruct((NPAD, cout), F32),
    )(pre, sums, g2, b2)


def kernel(x, edge_index,
           Ws0, Ws1, Ws2, Ws3, Ws4, Ws5, Ws6, Ws7, Ws8, Ws9, Ws10, Ws11,
           Wn0, Wn1, Wn2, Wn3, Wn4, Wn5, Wn6, Wn7, Wn8, Wn9, Wn10, Wn11,
           g0, g1, g2, g3, g4, g5, g6, g7, g8, g9,
           b0, b1, b2, b3, b4, b5, b6, b7, b8, b9):
    Ws = [Ws0, Ws1, Ws2, Ws3, Ws4, Ws5, Ws6, Ws7, Ws8, Ws9, Ws10, Ws11]
    Wn = [Wn0, Wn1, Wn2, Wn3, Wn4, Wn5, Wn6, Wn7, Wn8, Wn9, Wn10, Wn11]
    gl = [g0, g1, g2, g3, g4, g5, g6, g7, g8, g9]
    bl = [b0, b1, b2, b3, b4, b5, b6, b7, b8, b9]
    gl = [v.reshape(1, -1) for v in gl]
    bl = [v.reshape(1, -1) for v in bl]

    src = edge_index[0].astype(jnp.int32)
    dst = edge_index[1].astype(jnp.int32)
    srcp = jnp.concatenate([src, jnp.zeros((EPAD - E,), jnp.int32)])
    dstp = jnp.concatenate([dst, jnp.full((EPAD - E,), N, jnp.int32)])
    dst2d = dstp.reshape(EPAD // 128, 128)
    xp = jnp.pad(x, ((0, NPAD - N), (0, 0)))
    zc = {c: jnp.zeros((RPS, c), F32) for c in (8, 16)}

    sc_edge = {c: _make_sc(c, "edge") for c in (8, 16)}
    sc_cs = _make_sc(16, "cs")
    sc_cs4 = _make_sc(16, "cs4")
    bn_j = {ci: j for j, ci in enumerate([0, 1, 2, 3, 4, 5, 6, 8, 9, 10])}

    def conv(h, i, inv, extra_aug=False):
        cin, cout = Wn[i].shape
        if cout == 64:
            z = _t0(h, Wn[i], 4)
            p = sc_cs4(z, srcp, dst2d, zc[16])
            mode = "cs"
        elif cout == 32:
            z = _t0(h, Wn[i], 2)
            p = sc_cs(z, srcp, dst2d, zc[16])
            mode = "cs"
        else:
            if extra_aug:
                haug = jnp.concatenate([h, jnp.ones((NPAD, 1), F32)], axis=1)
                wn = jnp.zeros((cin + 1, 16), F32)
                wn = wn.at[:cin, :cout].set(Wn[i]).at[cin, 8].set(1.0)
                z = _t0(haug, wn, 1)
                p = sc_edge[16](z, srcp, dst2d, zc[16])
            else:
                cz = max(cout, 8)
                wn = (Wn[i] if cz == cout else
                      jnp.pad(Wn[i], ((0, 0), (0, cz - cout))))
                z = _t0(h, wn, 1)
                p = sc_edge[cz](z, srcp, dst2d, zc[cz])
            mode = "edge"
        if extra_aug:
            inv = _t3(p)
        want_bn = i in bn_j
        pre, s = _t1(p, inv, h, Ws[i], mode, want_bn)
        if want_bn:
            hn = _t2(pre, s, gl[bn_j[i]], bl[bn_j[i]])
        else:
            hn = pre
        return hn, inv

    h, inv = conv(xp, 0, None, extra_aug=True)
    for i in range(1, 4):
        h, _ = conv(h, i, inv)
    h3 = h
    r = h3
    for i in range(4, 8):
        r, _ = conv(r, i, inv)
    c_out = h3
    for i in range(8, 12):
        c_out, _ = conv(c_out, i, inv)
    return (c_out[:N], r[:N])


# speedup vs baseline: 15.3223x; 1.0817x over previous
import functools

import jax
import jax.numpy as jnp
from jax import lax
from jax.experimental import pallas as pl
from jax.experimental.pallas import tpu as pltpu
from jax.experimental.pallas import tpu_sc as plsc

N = 100000
NPAD = 100352
E = 1600000
EPAD = 1605632
CHUNK = 7168
CB = CHUNK // 128
RPS = NPAD // 16
BLK = 6272
NBLK = NPAD // BLK
F32 = jnp.float32


def _make_sc(c, mode):
    nchunks = 7 if mode == "edge" else 14
    npass = 2 if mode == "cs4" else 1
    nslab = {"edge": 2, "cs": 2, "cs4": 4}[mode]
    mesh = plsc.VectorSubcoreMesh(core_axis_name="core", subcore_axis_name="sub")

    def body(table, srcp, dst2d, zrs, out, src_v, dst_v, *rest):
        rows = list(rest[:8])
        acc = rest[8]
        sems = rest[9:]
        gsem = sems[:8]
        ssem = sems[8:]
        cr = lax.axis_index("core")
        sid = lax.axis_index("sub")
        my_rows = pl.multiple_of(sid * RPS, 128)
        if mode == "edge":
            base0 = (cr * 16 + sid) * (nchunks * CHUNK)
        else:
            base0 = sid * (nchunks * CHUNK)

        def run_pass(tbl, slab):
            pltpu.sync_copy(zrs, acc.at[pl.ds(my_rows, RPS)])
            plsc.subcore_barrier()

            def chunk_body(ch, carry):
                base = pl.multiple_of(base0 + ch * CHUNK, 128)
                pltpu.sync_copy(srcp.at[pl.ds(base, CHUNK)], src_v)
                pltpu.sync_copy(
                    dst2d.at[pl.ds(pl.multiple_of(base // 128, 8), CB)], dst_v)
                for b in range(8):
                    pltpu.async_copy(tbl.at[src_v.at[pl.ds(b * 128, 128)]],
                                     rows[b], gsem[b])

                def octet(j, c2):
                    for b in range(8):
                        bat = 8 * j + b
                        pltpu.make_async_copy(
                            tbl.at[src_v.at[pl.ds(bat * 128, 128)]],
                            rows[b], gsem[b]).wait()
                        pltpu.async_copy(rows[b], acc.at[dst_v.at[bat]],
                                         ssem[b], add=True)
                    for b in range(8):
                        pltpu.make_async_copy(rows[b],
                                              acc.at[dst_v.at[8 * j + b]],
                                              ssem[b]).wait()

                        @pl.when(j < CB // 8 - 1)
                        def _():
                            pltpu.async_copy(
                                tbl.at[src_v.at[pl.ds((8 * j + b + 8) * 128,
                                                      128)]],
                                rows[b], gsem[b])
                    return c2

                lax.fori_loop(0, CB // 8, octet, 0)
                return carry

            lax.fori_loop(0, nchunks, chunk_body, 0)
            plsc.subcore_barrier()
            pltpu.sync_copy(acc.at[pl.ds(my_rows, RPS)],
                            out.at[slab, pl.ds(my_rows, RPS)])

        if mode == "edge":
            run_pass(table, cr)
        elif mode == "cs":
            run_pass(table.at[cr], cr)
        else:
            for q in range(npass):
                run_pass(table.at[cr * 2 + q], cr * 2 + q)
                if q + 1 < npass:
                    plsc.subcore_barrier()

    return pl.kernel(
        body,
        out_type=jax.ShapeDtypeStruct((nslab, NPAD, c), F32),
        mesh=mesh,
        scratch_types=[
            pltpu.VMEM((CHUNK,), jnp.int32),
            pltpu.VMEM((CB, 128), jnp.int32),
        ] + [pltpu.VMEM((128, c), F32)] * 8 + [
            pltpu.VMEM_SHARED((NPAD, c), F32),
        ] + [pltpu.SemaphoreType.DMA] * 16,
        compiler_params=pltpu.CompilerParams(use_tc_tiling_on_sc=False),
    )


def _t0(h, Wn, nsplit):
    cin, cout = Wn.shape

    if nsplit > 1:
        def body(h_ref, w_ref, o_ref):
            hv = h_ref[...]
            w = w_ref[...]
            for q in range(nsplit):
                o_ref[q] = jnp.dot(hv, w[:, 16 * q:16 * (q + 1)],
                                   preferred_element_type=F32)

        return pl.pallas_call(
            body,
            grid=(NBLK,),
            in_specs=[pl.BlockSpec((BLK, cin), lambda i: (i, 0)),
                      pl.BlockSpec((cin, cout), lambda i: (0, 0))],
            out_specs=pl.BlockSpec((nsplit, BLK, 16), lambda i: (0, i, 0)),
            out_shape=jax.ShapeDtypeStruct((nsplit, NPAD, 16), F32),
        )(h, Wn)

    def body(h_ref, w_ref, o_ref):
        o_ref[...] = jnp.dot(h_ref[...], w_ref[...],
                             preferred_element_type=F32)

    return pl.pallas_call(
        body,
        grid=(NBLK,),
        in_specs=[pl.BlockSpec((BLK, cin), lambda i: (i, 0)),
                  pl.BlockSpec((cin, cout), lambda i: (0, 0))],
        out_specs=pl.BlockSpec((BLK, cout), lambda i: (i, 0)),
        out_shape=jax.ShapeDtypeStruct((NPAD, cout), F32),
    )(h, Wn)


def _t3(parts0):
    def body(p_ref, o_ref):
        p = p_ref[...]
        d = p[0, :, 8:9] + p[1, :, 8:9]
        o_ref[...] = 1.0 / jnp.maximum(d, 1.0)

    return pl.pallas_call(
        body,
        grid=(NBLK,),
        in_specs=[pl.BlockSpec((2, BLK, 16), lambda i: (0, i, 0))],
        out_specs=pl.BlockSpec((BLK, 1), lambda i: (i, 0)),
        out_shape=jax.ShapeDtypeStruct((NPAD, 1), F32),
    )(parts0)


def _t1(parts, inv, h, Ws, mode, stats):
    cin, cout = Ws.shape
    nslab = parts.shape[0]
    ce = parts.shape[2]

    def body(p_ref, inv_ref, h_ref, ws_ref, *orefs):
        i = pl.program_id(0)
        p = p_ref[...]
        invv = inv_ref[...]
        if mode == "edge":
            pre = (p[0][:, :cout] + p[1][:, :cout]) * invv
        else:
            pre = jnp.concatenate([p[q] for q in range(nslab)], axis=1) * invv
        pre = pre + jnp.dot(h_ref[...], ws_ref[...],
                            preferred_element_type=F32)
        orefs[0][...] = pre
        if stats:
            s_ref = orefs[1]

            @pl.when(i == 0)
            def _():
                s_ref[...] = jnp.zeros_like(s_ref)

            ridx = i * BLK + lax.broadcasted_iota(jnp.int32, (BLK, 1), 0)
            m = (ridx < N).astype(F32)
            pm = pre * m
            s_ref[0:1, :] += jnp.sum(pm, axis=0, keepdims=True)
            s_ref[1:2, :] += jnp.sum(pre * pm, axis=0, keepdims=True)

    in_specs = [pl.BlockSpec((nslab, BLK, ce), lambda i: (0, i, 0)),
                pl.BlockSpec((BLK, 1), lambda i: (i, 0)),
                pl.BlockSpec((BLK, cin), lambda i: (i, 0)),
                pl.BlockSpec((cin, cout), lambda i: (0, 0))]
    out_specs = [pl.BlockSpec((BLK, cout), lambda i: (i, 0))]
    out_shape = [jax.ShapeDtypeStruct((NPAD, cout), F32)]
    if stats:
        out_specs.append(pl.BlockSpec((2, cout), lambda i: (0, 0)))
        out_shape.append(jax.ShapeDtypeStruct((2, cout), F32))

    res = pl.pallas_call(
        body,
        grid=(NBLK,),
        in_specs=in_specs,
        out_specs=tuple(out_specs) if stats else out_specs[0],
        out_shape=tuple(out_shape) if stats else out_shape[0],
    )(parts, inv, h, Ws)
    return res if stats else (res, None)


def _t2(pre, sums, g2, b2):
    cout = pre.shape[1]

    def body(pre_ref, s_ref, g_ref, b_ref, o_ref):
        s = s_ref[...]
        mu = s[0:1, :] * (1.0 / N)
        var = s[1:2, :] * (1.0 / N) - mu * mu
        scale = g_ref[...] * lax.rsqrt(var + 1e-5)
        shift = b_ref[...] - mu * scale
        o_ref[...] = jnp.maximum(pre_ref[...] * scale + shift, 0.0)

    return pl.pallas_call(
        body,
        grid=(NBLK,),
        in_specs=[pl.BlockSpec((BLK, cout), lambda i: (i, 0)),
                  pl.BlockSpec((2, cout), lambda i: (0, 0)),
                  pl.BlockSpec((1, cout), lambda i: (0, 0)),
                  pl.BlockSpec((1, cout), lambda i: (0, 0))],
        out_specs=pl.BlockSpec((BLK, cout), lambda i: (i, 0)),
        out_shape=jax.ShapeDtypeStruct((NPAD, cout), F32),
    )(pre, sums, g2, b2)


def kernel(x, edge_index,
           Ws0, Ws1, Ws2, Ws3, Ws4, Ws5, Ws6, Ws7, Ws8, Ws9, Ws10, Ws11,
           Wn0, Wn1, Wn2, Wn3, Wn4, Wn5, Wn6, Wn7, Wn8, Wn9, Wn10, Wn11,
           g0, g1, g2, g3, g4, g5, g6, g7, g8, g9,
           b0, b1, b2, b3, b4, b5, b6, b7, b8, b9):
    Ws = [Ws0, Ws1, Ws2, Ws3, Ws4, Ws5, Ws6, Ws7, Ws8, Ws9, Ws10, Ws11]
    Wn = [Wn0, Wn1, Wn2, Wn3, Wn4, Wn5, Wn6, Wn7, Wn8, Wn9, Wn10, Wn11]
    gl = [g0, g1, g2, g3, g4, g5, g6, g7, g8, g9]
    bl = [b0, b1, b2, b3, b4, b5, b6, b7, b8, b9]
    gl = [v.reshape(1, -1) for v in gl]
    bl = [v.reshape(1, -1) for v in bl]

    src = edge_index[0].astype(jnp.int32)
    dst = edge_index[1].astype(jnp.int32)
    srcp = jnp.concatenate([src, jnp.zeros((EPAD - E,), jnp.int32)])
    dstp = jnp.concatenate([dst, jnp.full((EPAD - E,), N, jnp.int32)])
    dst2d = dstp.reshape(EPAD // 128, 128)
    xp = jnp.pad(x, ((0, NPAD - N), (0, 0)))
    zc = {c: jnp.zeros((RPS, c), F32) for c in (8, 16)}

    sc_edge = {c: _make_sc(c, "edge") for c in (8, 16)}
    sc_cs = _make_sc(16, "cs")
    sc_cs4 = _make_sc(16, "cs4")
    bn_j = {ci: j for j, ci in enumerate([0, 1, 2, 3, 4, 5, 6, 8, 9, 10])}

    def conv(h, i, inv, extra_aug=False):
        cin, cout = Wn[i].shape
        if cout == 64:
            z = _t0(h, Wn[i], 4)
            p = sc_cs4(z, srcp, dst2d, zc[16])
            mode = "cs"
        elif cout == 32:
            z = _t0(h, Wn[i], 2)
            p = sc_cs(z, srcp, dst2d, zc[16])
            mode = "cs"
        else:
            if extra_aug:
                haug = jnp.concatenate([h, jnp.ones((NPAD, 1), F32)], axis=1)
                wn = jnp.zeros((cin + 1, 16), F32)
                wn = wn.at[:cin, :cout].set(Wn[i]).at[cin, 8].set(1.0)
                z = _t0(haug, wn, 1)
                p = sc_edge[16](z, srcp, dst2d, zc[16])
            else:
                cz = max(cout, 8)
                wn = (Wn[i] if cz == cout else
                      jnp.pad(Wn[i], ((0, 0), (0, cz - cout))))
                z = _t0(h, wn, 1)
                p = sc_edge[cz](z, srcp, dst2d, zc[cz])
            mode = "edge"
        if extra_aug:
            inv = _t3(p)
        want_bn = i in bn_j
        pre, s = _t1(p, inv, h, Ws[i], mode, want_bn)
        if want_bn:
            hn = _t2(pre, s, gl[bn_j[i]], bl[bn_j[i]])
        else:
            hn = pre
        return hn, inv

    h, inv = conv(xp, 0, None, extra_aug=True)
    for i in range(1, 4):
        h, _ = conv(h, i, inv)
    h3 = h
    r = h3
    for i in range(4, 8):
        r, _ = conv(r, i, inv)
    c_out = h3
    for i in range(8, 12):
        c_out, _ = conv(c_out, i, inv)
    return (c_out[:N], r[:N])
